# baseline (speedup 1.0000x reference)
import functools

import jax
import jax.numpy as jnp
from jax.experimental import pallas as pl
from jax.experimental.pallas import tpu as pltpu

LANES = 128
GROUP = 32
NPACK = LANES // GROUP
CL = GROUP - 1

_SIZES = dict(ew1=128, eb1=8, ew2=128, eb2=8, ew3=128, eb3=8, ew4=128, eb4=8,
              pw=128, dw1b=128, dw1d=128, db1=8, dw2=128, dw3=128,
              dw4=128, db4=8)
_OFFS = {}
_off = 0
for _k, _n in _SIZES.items():
    _OFFS[_k] = _off
    _off += _n
SLAB_ROWS = _off

_BF_NAMES = ("dw2", "dw3", "dw4", "ew1", "ew2", "ew3", "ew4", "dw1b")
_BF = {n: i * LANES for i, n in enumerate(_BF_NAMES)}
BF_ROWS = len(_BF_NAMES) * LANES


def _prep_body(degree, t84_ref, ew1_ref, eb1_ref, ew2_ref, eb2_ref, ew3_ref,
               eb3_ref, ew4_ref, eb4_ref, pw_ref, dw1_ref, db1_ref, dw2_ref,
               db2_ref, dw3_ref, db3_ref, dw4_ref, db4_ref,
               slab_ref, tp_ref, sbf_ref):
    f32 = jnp.float32
    slab_ref[...] = jnp.zeros(slab_ref.shape, f32)

    def put(name, w_ref, rows=None, cols=None, r0=0, c0=0):
        w = w_ref[...].astype(f32)
        o = _OFFS[name]
        slab_ref[o + r0:o + r0 + w.shape[0], c0:c0 + w.shape[1]] = w

    put("ew1", ew1_ref)
    put("eb1", eb1_ref)
    put("ew2", ew2_ref)
    put("eb2", eb2_ref)
    put("ew3", ew3_ref)
    put("eb3", eb3_ref)
    put("ew4", ew4_ref)
    put("eb4", eb4_ref)

    pwv = pw_ref[...].astype(f32)
    dw1v = dw1_ref[...].astype(f32)
    dw2v = dw2_ref[...].astype(f32)
    dw3v = dw3_ref[...].astype(f32)
    dw4v = dw4_ref[...].astype(f32)
    for j in range(NPACK):
        c = j * GROUP
        slab_ref[_OFFS["pw"] + c:_OFFS["pw"] + c + pwv.shape[0],
                 c:c + pwv.shape[1]] = pwv
        slab_ref[_OFFS["dw1b"]:_OFFS["dw1b"] + dw1v.shape[0],
                 c:c + dw1v.shape[1]] = dw1v
        slab_ref[_OFFS["dw1d"] + c:_OFFS["dw1d"] + c + dw1v.shape[0],
                 c:c + dw1v.shape[1]] = dw1v
        slab_ref[_OFFS["dw2"] + c:_OFFS["dw2"] + c + dw2v.shape[0],
                 c:c + dw2v.shape[1]] = dw2v
        slab_ref[_OFFS["dw3"] + c:_OFFS["dw3"] + c + dw3v.shape[0],
                 c:c + dw3v.shape[1]] = dw3v
        q = dw4v.shape[1]
        slab_ref[_OFFS["dw4"] + c:_OFFS["dw4"] + c + dw4v.shape[0],
                 j * q:(j + 1) * q] = dw4v

    db1v = db1_ref[...].astype(f32)
    db2v = db2_ref[...].astype(f32)
    db3v = db3_ref[...].astype(f32)
    db4v = db4_ref[...].astype(f32)

    n1 = db1v.shape[1]
    piece1 = jnp.concatenate(
        [db1v, jnp.zeros((1, GROUP - n1 - 1), f32), jnp.ones((1, 1), f32)], axis=1)
    slab_ref[_OFFS["db1"]:_OFFS["db1"] + 1, :] = jnp.concatenate(
        [piece1] * NPACK, axis=1)

    n2 = db2v.shape[1]
    piece2 = jnp.concatenate(
        [db2v, jnp.zeros((1, GROUP - n2 - 1), f32), jnp.ones((1, 1), f32)], axis=1)
    q = db4v.shape[1]
    row4 = jnp.concatenate([db4v] * NPACK, axis=1)
    for j in range(NPACK):
        c = j * GROUP
        slab_ref[_OFFS["dw2"] + c + CL:_OFFS["dw2"] + c + CL + 1,
                 c:c + GROUP] = piece2
        slab_ref[_OFFS["dw3"] + c + CL:_OFFS["dw3"] + c + CL + 1,
                 c:c + GROUP] = db3v
    slab_ref[_OFFS["db4"]:_OFFS["db4"] + 1, 0:NPACK * q] = row4

    t84 = t84_ref[...].astype(f32)
    G = t84.shape[0]
    ri = jax.lax.broadcasted_iota(jnp.int32, (NPACK, LANES), 0)
    li = jax.lax.broadcasted_iota(jnp.int32, (NPACK, LANES), 1)
    E = (li // GROUP == ri).astype(f32)
    tvals = jnp.dot(t84, E, preferred_element_type=f32)
    k = jax.lax.broadcasted_iota(jnp.int32, (G, LANES), 1) % GROUP
    acc = jnp.zeros((G, LANES), f32)
    p = tvals
    for d in range(degree):
        acc = acc + jnp.where(k == d, p, 0.0)
        p = p * tvals
    tp_ref[...] = acc

    for i, name in enumerate(_BF_NAMES):
        sbf_ref[i * LANES:(i + 1) * LANES, :] = slab_ref[
            _OFFS[name]:_OFFS[name] + LANES, :].astype(jnp.bfloat16)


def _main_body(tb, G, Q, x0_ref, tp_ref, slab_ref, sbf_ref, out_ref):
    P = LANES
    f32 = jnp.float32

    def W(o):
        return slab_ref[o:o + P, :]

    def Bv(o):
        return slab_ref[o:o + 1, :]

    o = _OFFS
    bf16 = jnp.bfloat16

    def Wb(name):
        ob = _BF[name]
        return sbf_ref[ob:ob + P, :]

    h = jnp.concatenate(
        [x0_ref[...].astype(bf16), jnp.zeros((tb, P - Q), bf16)], axis=1)
    h = jnp.maximum(jnp.dot(h, Wb("ew1"), preferred_element_type=f32) + Bv(o["eb1"]), 0.0).astype(bf16)
    h = jnp.maximum(jnp.dot(h, Wb("ew2"), preferred_element_type=f32) + Bv(o["eb2"]), 0.0).astype(bf16)
    h = jnp.maximum(jnp.dot(h, Wb("ew3"), preferred_element_type=f32) + Bv(o["eb3"]), 0.0).astype(bf16)
    z0 = jnp.tanh(jnp.dot(h, Wb("ew4"), preferred_element_type=f32) + Bv(o["eb4"])).astype(bf16)

    a1 = jnp.dot(z0, Wb("dw1b"), preferred_element_type=f32)

    poly = jnp.dot(tp_ref[...], W(o["pw"]), preferred_element_type=f32)
    c1 = jnp.dot(poly, W(o["dw1d"]), preferred_element_type=f32) + Bv(o["db1"])

    h = jnp.maximum(c1[:, None, :] + a1[None, :, :], 0.0)
    h = h.reshape(G * tb, P).astype(bf16)

    h = jnp.maximum(
        jnp.dot(h, Wb("dw2"), preferred_element_type=f32), 0.0).astype(bf16)
    h = jnp.maximum(
        jnp.dot(h, Wb("dw3"), preferred_element_type=f32), 0.0).astype(bf16)
    y = jnp.dot(h, Wb("dw4"), preferred_element_type=f32) + Bv(o["db4"])

    yr = y.reshape(G, tb, P)
    w = NPACK * Q
    out_ref[...] = jnp.tanh(jnp.concatenate(
        [yr[g, :, :w] for g in range(G)], axis=-1)).astype(out_ref.dtype)


def kernel(x, t_range, ew1, eb1, ew2, eb2, ew3, eb3, ew4, eb4, pw,
           dw1, db1, dw2, db2, dw3, db3, dw4, db4):
    B, T, Q = x.shape
    P = LANES
    G = T // NPACK
    degree = pw.shape[0]

    tb = 4096 if B >= 8192 else max(8, min(B, 128))
    n_tiles = -(-B // tb)
    Bp = n_tiles * tb

    xk = x[:, 0, :].astype(jnp.float32)
    if Bp != B:
        xk = jnp.zeros((Bp, Q), jnp.float32).at[:B].set(xk)

    t84 = t_range.astype(jnp.float32).reshape(G, NPACK)

    slab, tp, sbf = pl.pallas_call(
        functools.partial(_prep_body, degree),
        out_shape=[
            jax.ShapeDtypeStruct((SLAB_ROWS, P), jnp.float32),
            jax.ShapeDtypeStruct((G, P), jnp.float32),
            jax.ShapeDtypeStruct((BF_ROWS, P), jnp.bfloat16),
        ],
    )(t84, ew1, eb1, ew2, eb2, ew3, eb3, ew4, eb4, pw,
      dw1, db1, dw2, db2, dw3, db3, dw4, db4)

    body = functools.partial(_main_body, tb, G, Q)

    out = pl.pallas_call(
        body,
        out_shape=jax.ShapeDtypeStruct((Bp, P), jnp.float32),
        grid=(n_tiles,),
        in_specs=[
            pl.BlockSpec((tb, Q), lambda b: (b, 0)),
            pl.BlockSpec((G, P), lambda b: (0, 0)),
            pl.BlockSpec((SLAB_ROWS, P), lambda b: (0, 0)),
            pl.BlockSpec((BF_ROWS, P), lambda b: (0, 0)),
        ],
        out_specs=pl.BlockSpec((tb, P), lambda b: (b, 0)),
        compiler_params=pltpu.CompilerParams(dimension_semantics=("parallel",)),
    )(xk, tp, slab, sbf)

    return out[:B].reshape(B, T, Q)


# speedup vs baseline: 1.0036x; 1.0036x over previous
import functools

import jax
import jax.numpy as jnp
from jax.experimental import pallas as pl
from jax.experimental.pallas import tpu as pltpu

LANES = 128
GROUP = 32
NPACK = LANES // GROUP
CL = GROUP - 1

_SIZES = dict(ew1=128, eb1=8, ew2=128, eb2=8, ew3=128, eb3=8, ew4=128, eb4=8,
              pw=128, dw1b=128, dw1d=128, db1=8, dw2=128, dw3=128,
              dw4=128, db4=8, db4f=8)
_OFFS = {}
_off = 0
for _k, _n in _SIZES.items():
    _OFFS[_k] = _off
    _off += _n
SLAB_ROWS = _off

_BF_NAMES = ("dw2", "dw3", "dw4", "ew1", "ew2", "ew3", "ew4", "dw1b")
_BF = {n: i * LANES for i, n in enumerate(_BF_NAMES)}
BF_ROWS = len(_BF_NAMES) * LANES


def _prep_body(degree, t84_ref, ew1_ref, eb1_ref, ew2_ref, eb2_ref, ew3_ref,
               eb3_ref, ew4_ref, eb4_ref, pw_ref, dw1_ref, db1_ref, dw2_ref,
               db2_ref, dw3_ref, db3_ref, dw4_ref, db4_ref,
               slab_ref, tp_ref, sbf_ref):
    f32 = jnp.float32
    slab_ref[...] = jnp.zeros(slab_ref.shape, f32)

    def put(name, w_ref, rows=None, cols=None, r0=0, c0=0):
        w = w_ref[...].astype(f32)
        o = _OFFS[name]
        slab_ref[o + r0:o + r0 + w.shape[0], c0:c0 + w.shape[1]] = w

    put("ew1", ew1_ref)
    put("eb1", eb1_ref)
    put("ew2", ew2_ref)
    put("eb2", eb2_ref)
    put("ew3", ew3_ref)
    put("eb3", eb3_ref)
    put("ew4", ew4_ref)
    put("eb4", eb4_ref)

    pwv = pw_ref[...].astype(f32)
    dw1v = dw1_ref[...].astype(f32)
    dw2v = dw2_ref[...].astype(f32)
    dw3v = dw3_ref[...].astype(f32)
    dw4v = dw4_ref[...].astype(f32)
    for j in range(NPACK):
        c = j * GROUP
        slab_ref[_OFFS["pw"] + c:_OFFS["pw"] + c + pwv.shape[0],
                 c:c + pwv.shape[1]] = pwv
        slab_ref[_OFFS["dw1b"]:_OFFS["dw1b"] + dw1v.shape[0],
                 c:c + dw1v.shape[1]] = dw1v
        slab_ref[_OFFS["dw1d"] + c:_OFFS["dw1d"] + c + dw1v.shape[0],
                 c:c + dw1v.shape[1]] = dw1v
        slab_ref[_OFFS["dw2"] + c:_OFFS["dw2"] + c + dw2v.shape[0],
                 c:c + dw2v.shape[1]] = dw2v
        slab_ref[_OFFS["dw3"] + c:_OFFS["dw3"] + c + dw3v.shape[0],
                 c:c + dw3v.shape[1]] = dw3v
        q = dw4v.shape[1]
        slab_ref[_OFFS["dw4"] + c:_OFFS["dw4"] + c + dw4v.shape[0],
                 j * q:(j + 1) * q] = dw4v

    db1v = db1_ref[...].astype(f32)
    db2v = db2_ref[...].astype(f32)
    db3v = db3_ref[...].astype(f32)
    db4v = db4_ref[...].astype(f32)

    n1 = db1v.shape[1]
    piece1 = jnp.concatenate(
        [db1v, jnp.zeros((1, GROUP - n1 - 1), f32), jnp.ones((1, 1), f32)], axis=1)
    slab_ref[_OFFS["db1"]:_OFFS["db1"] + 1, :] = jnp.concatenate(
        [piece1] * NPACK, axis=1)

    n2 = db2v.shape[1]
    piece2 = jnp.concatenate(
        [db2v, jnp.zeros((1, GROUP - n2 - 1), f32), jnp.ones((1, 1), f32)], axis=1)
    q = db4v.shape[1]
    row4 = jnp.concatenate([db4v] * NPACK, axis=1)
    for j in range(NPACK):
        c = j * GROUP
        slab_ref[_OFFS["dw2"] + c + CL:_OFFS["dw2"] + c + CL + 1,
                 c:c + GROUP] = piece2
        slab_ref[_OFFS["dw3"] + c + CL:_OFFS["dw3"] + c + CL + 1,
                 c:c + GROUP] = db3v
    slab_ref[_OFFS["db4"]:_OFFS["db4"] + 1, 0:NPACK * q] = row4
    slab_ref[_OFFS["db4f"]:_OFFS["db4f"] + 1, :] = jnp.concatenate(
        [row4] * (LANES // (NPACK * q)), axis=1)

    t84 = t84_ref[...].astype(f32)
    G = t84.shape[0]
    ri = jax.lax.broadcasted_iota(jnp.int32, (NPACK, LANES), 0)
    li = jax.lax.broadcasted_iota(jnp.int32, (NPACK, LANES), 1)
    E = (li // GROUP == ri).astype(f32)
    tvals = jnp.dot(t84, E, preferred_element_type=f32)
    k = jax.lax.broadcasted_iota(jnp.int32, (G, LANES), 1) % GROUP
    acc = jnp.zeros((G, LANES), f32)
    p = tvals
    for d in range(degree):
        acc = acc + jnp.where(k == d, p, 0.0)
        p = p * tvals
    tp_ref[...] = acc

    for i, name in enumerate(_BF_NAMES):
        sbf_ref[i * LANES:(i + 1) * LANES, :] = slab_ref[
            _OFFS[name]:_OFFS[name] + LANES, :].astype(jnp.bfloat16)


def _main_body(tb, G, Q, x0_ref, tp_ref, slab_ref, sbf_ref, out_ref):
    P = LANES
    f32 = jnp.float32

    def W(o):
        return slab_ref[o:o + P, :]

    def Bv(o):
        return slab_ref[o:o + 1, :]

    o = _OFFS
    bf16 = jnp.bfloat16

    def Wb(name):
        ob = _BF[name]
        return sbf_ref[ob:ob + P, :]

    h = jnp.concatenate(
        [x0_ref[...].astype(bf16), jnp.zeros((tb, P - Q), bf16)], axis=1)
    h = jnp.maximum(jnp.dot(h, Wb("ew1"), preferred_element_type=f32) + Bv(o["eb1"]), 0.0).astype(bf16)
    h = jnp.maximum(jnp.dot(h, Wb("ew2"), preferred_element_type=f32) + Bv(o["eb2"]), 0.0).astype(bf16)
    h = jnp.maximum(jnp.dot(h, Wb("ew3"), preferred_element_type=f32) + Bv(o["eb3"]), 0.0).astype(bf16)
    z0 = jnp.tanh(jnp.dot(h, Wb("ew4"), preferred_element_type=f32) + Bv(o["eb4"])).astype(bf16)

    a1 = jnp.dot(z0, Wb("dw1b"), preferred_element_type=f32)

    poly = jnp.dot(tp_ref[...], W(o["pw"]), preferred_element_type=f32)
    c1 = jnp.dot(poly, W(o["dw1d"]), preferred_element_type=f32) + Bv(o["db1"])

    h = jnp.maximum(c1[:, None, :] + a1[None, :, :], 0.0)
    h = h.reshape(G * tb, P).astype(bf16)

    h = jnp.maximum(
        jnp.dot(h, Wb("dw2"), preferred_element_type=f32), 0.0).astype(bf16)
    h = jnp.maximum(
        jnp.dot(h, Wb("dw3"), preferred_element_type=f32), 0.0).astype(bf16)
    y = jnp.dot(h, Wb("dw4"), preferred_element_type=f32)

    yr = y.reshape(G, tb, P)
    w = NPACK * Q
    folded = jnp.concatenate([yr[g, :, :w] for g in range(G)], axis=-1)
    out_ref[...] = jnp.tanh(folded + Bv(o["db4f"])).astype(out_ref.dtype)


def kernel(x, t_range, ew1, eb1, ew2, eb2, ew3, eb3, ew4, eb4, pw,
           dw1, db1, dw2, db2, dw3, db3, dw4, db4):
    B, T, Q = x.shape
    P = LANES
    G = T // NPACK
    degree = pw.shape[0]

    tb = 4096 if B >= 8192 else max(8, min(B, 128))
    n_tiles = -(-B // tb)
    Bp = n_tiles * tb

    xk = x[:, 0, :].astype(jnp.float32)
    if Bp != B:
        xk = jnp.zeros((Bp, Q), jnp.float32).at[:B].set(xk)

    t84 = t_range.astype(jnp.float32).reshape(G, NPACK)

    slab, tp, sbf = pl.pallas_call(
        functools.partial(_prep_body, degree),
        out_shape=[
            jax.ShapeDtypeStruct((SLAB_ROWS, P), jnp.float32),
            jax.ShapeDtypeStruct((G, P), jnp.float32),
            jax.ShapeDtypeStruct((BF_ROWS, P), jnp.bfloat16),
        ],
    )(t84, ew1, eb1, ew2, eb2, ew3, eb3, ew4, eb4, pw,
      dw1, db1, dw2, db2, dw3, db3, dw4, db4)

    body = functools.partial(_main_body, tb, G, Q)

    out = pl.pallas_call(
        body,
        out_shape=jax.ShapeDtypeStruct((Bp, P), jnp.float32),
        grid=(n_tiles,),
        in_specs=[
            pl.BlockSpec((tb, Q), lambda b: (b, 0)),
            pl.BlockSpec((G, P), lambda b: (0, 0)),
            pl.BlockSpec((SLAB_ROWS, P), lambda b: (0, 0)),
            pl.BlockSpec((BF_ROWS, P), lambda b: (0, 0)),
        ],
        out_specs=pl.BlockSpec((tb, P), lambda b: (b, 0)),
        compiler_params=pltpu.CompilerParams(dimension_semantics=("parallel",)),
    )(xk, tp, slab, sbf)

    return out[:B].reshape(B, T, Q)
